# baseline (speedup 1.0000x reference)
import jax
import jax.numpy as jnp
from jax.experimental import pallas as pl
from jax.experimental.pallas import tpu as pltpu

_B, _N, _F0 = 4, 1024, 11
_H, _F1, _F2 = 6, 32, 64
_D1 = _H * _F1
_D2 = _H * _F2


def _dotT(a, b):
    return jax.lax.dot_general(a, b, (((1,), (1,)), ((), ())),
                               preferred_element_type=jnp.float32)


def _dotN(a, b):
    return jax.lax.dot_general(a, b, (((0,), (0,)), ((), ())),
                               preferred_element_type=jnp.float32)


def _gat_head(bias, Wh, src_col, dst_row):
    e = src_col + dst_row
    e = jnp.minimum(jnp.maximum(e, 0.2 * e) + bias.astype(jnp.float32), 86.0)
    p = jnp.exp2(e).astype(jnp.bfloat16)
    ones = jnp.ones((_N, 1), jnp.bfloat16)
    num = jnp.dot(p, jnp.concatenate([Wh.astype(jnp.bfloat16), ones], axis=1),
                  preferred_element_type=jnp.float32)
    o = num[:, :-1] / num[:, -1:]
    return jnp.where(o > 0, o, jnp.exp(jnp.minimum(o, 0.0)) - 1.0)


def _fused(h_ref, adj_ref, W1_ref, as1_ref, ad1_ref, W2_ref, as2_ref,
           ad2_ref, Ws1_ref, Ws2_ref, WdR_ref, out_ref,
           bias_ref, x1_ref, x2_ref):
    phase = pl.program_id(1)
    k = pl.program_id(2)

    @pl.when(jnp.logical_and(phase == 0, k == 0))
    def _():
        bias_ref[...] = jnp.where(adj_ref[0] > 0.5, jnp.float32(0.0),
                                  jnp.float32(-1e9)).astype(jnp.bfloat16)

    @pl.when(phase == 0)
    def _():
        Wh = jnp.dot(h_ref[0], W1_ref[k], preferred_element_type=jnp.float32)
        src_col = jnp.dot(Wh, as1_ref[k], preferred_element_type=jnp.float32)
        dst_row = _dotT(ad1_ref[k], Wh)
        x1_ref[k] = _gat_head(bias_ref[...], Wh, src_col, dst_row)

    @pl.when(phase == 1)
    def _():
        Wh = jnp.dot(x1_ref[0], W2_ref[k, 0:_F1, :],
                     preferred_element_type=jnp.float32)
        for j in range(1, _H):
            Wh += jnp.dot(x1_ref[j], W2_ref[k, j * _F1:(j + 1) * _F1, :],
                          preferred_element_type=jnp.float32)
        src_col = jnp.dot(Wh, as2_ref[k], preferred_element_type=jnp.float32)
        dst_row = _dotT(ad2_ref[k], Wh)
        x2_ref[k] = _gat_head(bias_ref[...], Wh, src_col, dst_row)

    @pl.when(jnp.logical_and(phase == 1, k == _H - 1))
    def _():
        u = jnp.dot(x2_ref[0], Ws1_ref[0:_F2, :],
                    preferred_element_type=jnp.float32)
        for j in range(1, _H):
            u += jnp.dot(x2_ref[j], Ws1_ref[j * _F2:(j + 1) * _F2, :],
                         preferred_element_type=jnp.float32)
        u = jnp.tanh(u)
        scores = _dotT(u, Ws2_ref[...])
        m = jnp.max(scores, axis=0, keepdims=True)
        p = jnp.exp(scores - m)
        A = p / jnp.sum(p, axis=0, keepdims=True)
        val = jnp.float32(0.0)
        for j in range(_H):
            Mj = _dotN(A, x2_ref[j])
            val += jnp.sum(Mj * WdR_ref[:, j * _F2:(j + 1) * _F2])
        out_ref[...] = jnp.zeros((1, 8, 128), jnp.float32) + val


def kernel(h, adj, W1, a_src1, a_dst1, W2, a_src2, a_dst2, Ws1, Ws2, Wd, bd):
    B, N, F0 = h.shape
    c = jnp.float32(1.4426950408889634)
    as1 = a_src1[:, :, None] * c
    ad1 = a_dst1[:, None, :] * c
    as2 = a_src2[:, :, None] * c
    ad2 = a_dst2[:, None, :] * c
    WdR = Wd.reshape(_H, _D2)

    out = pl.pallas_call(
        _fused,
        grid=(B, 2, _H),
        in_specs=[
            pl.BlockSpec((1, N, F0), lambda b, p, k: (b, 0, 0)),
            pl.BlockSpec((1, N, N), lambda b, p, k: (b, 0, 0)),
            pl.BlockSpec((_H, F0, _F1), lambda b, p, k: (0, 0, 0)),
            pl.BlockSpec((_H, _F1, 1), lambda b, p, k: (0, 0, 0)),
            pl.BlockSpec((_H, 1, _F1), lambda b, p, k: (0, 0, 0)),
            pl.BlockSpec((_H, _D1, _F2), lambda b, p, k: (0, 0, 0)),
            pl.BlockSpec((_H, _F2, 1), lambda b, p, k: (0, 0, 0)),
            pl.BlockSpec((_H, 1, _F2), lambda b, p, k: (0, 0, 0)),
            pl.BlockSpec((_D2, _D2), lambda b, p, k: (0, 0)),
            pl.BlockSpec((_H, _D2), lambda b, p, k: (0, 0)),
            pl.BlockSpec((_H, _D2), lambda b, p, k: (0, 0)),
        ],
        out_specs=pl.BlockSpec((1, 8, 128), lambda b, p, k: (b, 0, 0)),
        out_shape=jax.ShapeDtypeStruct((B, 8, 128), jnp.float32),
        compiler_params=pltpu.CompilerParams(
            dimension_semantics=("parallel", "arbitrary", "arbitrary")),
        scratch_shapes=[
            pltpu.VMEM((N, N), jnp.bfloat16),
            pltpu.VMEM((_H, N, _F1), jnp.float32),
            pltpu.VMEM((_H, N, _F2), jnp.float32),
        ],
    )(h, adj, W1, as1, ad1, W2, as2, ad2, Ws1, Ws2, WdR)
    return out[:, 0, 0] + bd[0]


# speedup vs baseline: 1.1258x; 1.1258x over previous
import jax
import jax.numpy as jnp
from jax.experimental import pallas as pl
from jax.experimental.pallas import tpu as pltpu

_B, _N, _F0 = 4, 1024, 11
_H, _F1, _F2 = 6, 32, 64
_D1 = _H * _F1
_D2 = _H * _F2


def _dotT(a, b):
    return jax.lax.dot_general(a, b, (((1,), (1,)), ((), ())),
                               preferred_element_type=jnp.float32)


def _dotN(a, b):
    return jax.lax.dot_general(a, b, (((0,), (0,)), ((), ())),
                               preferred_element_type=jnp.float32)


def _gat_head(bias, Wh, src_col, dst_row):
    e = src_col + dst_row
    e = jnp.minimum(jnp.maximum(e, 0.2 * e) + bias, 86.0)
    p = jnp.exp2(e).astype(jnp.bfloat16)
    ones = jnp.ones((_N, 1), jnp.bfloat16)
    num = jnp.dot(p, jnp.concatenate([Wh.astype(jnp.bfloat16), ones], axis=1),
                  preferred_element_type=jnp.float32)
    o = num[:, :-1] / num[:, -1:]
    return jnp.where(o > 0, o, jnp.exp(jnp.minimum(o, 0.0)) - 1.0)


def _fused(h_ref, adj_ref, W1_ref, as1_ref, ad1_ref, W2_ref, as2_ref,
           ad2_ref, Ws1_ref, Ws2_ref, WdR_ref, out_ref,
           bias_ref, x1_ref, x2_ref):
    phase = pl.program_id(1)
    k2 = pl.program_id(2)

    @pl.when(jnp.logical_and(phase == 0, k2 == 0))
    def _():
        bias_ref[...] = jnp.where(adj_ref[0] > 0.5, jnp.float32(0.0),
                                  jnp.float32(-1e9))

    @pl.when(phase == 0)
    def _():
        for dk in range(2):
            k = k2 * 2 + dk
            Wh = jnp.dot(h_ref[0], W1_ref[k],
                         preferred_element_type=jnp.float32)
            src_col = jnp.dot(Wh, as1_ref[k],
                              preferred_element_type=jnp.float32)
            dst_row = _dotT(ad1_ref[k], Wh)
            x1_ref[k] = _gat_head(bias_ref[...], Wh, src_col, dst_row)

    @pl.when(phase == 1)
    def _():
        for dk in range(2):
            k = k2 * 2 + dk
            Wh = jnp.dot(x1_ref[0], W2_ref[k, 0:_F1, :],
                         preferred_element_type=jnp.float32)
            for j in range(1, _H):
                Wh += jnp.dot(x1_ref[j], W2_ref[k, j * _F1:(j + 1) * _F1, :],
                              preferred_element_type=jnp.float32)
            src_col = jnp.dot(Wh, as2_ref[k],
                              preferred_element_type=jnp.float32)
            dst_row = _dotT(ad2_ref[k], Wh)
            x2_ref[k] = _gat_head(bias_ref[...], Wh, src_col, dst_row)

    @pl.when(jnp.logical_and(phase == 1, k2 == _H // 2 - 1))
    def _():
        u = jnp.dot(x2_ref[0], Ws1_ref[0:_F2, :],
                    preferred_element_type=jnp.float32)
        for j in range(1, _H):
            u += jnp.dot(x2_ref[j], Ws1_ref[j * _F2:(j + 1) * _F2, :],
                         preferred_element_type=jnp.float32)
        u = jnp.tanh(u)
        scores = _dotT(u, Ws2_ref[...])
        m = jnp.max(scores, axis=0, keepdims=True)
        p = jnp.exp(scores - m)
        A = p / jnp.sum(p, axis=0, keepdims=True)
        val = jnp.float32(0.0)
        for j in range(_H):
            Mj = _dotN(A, x2_ref[j])
            val += jnp.sum(Mj * WdR_ref[:, j * _F2:(j + 1) * _F2])
        out_ref[...] = jnp.zeros((1, 8, 128), jnp.float32) + val


def kernel(h, adj, W1, a_src1, a_dst1, W2, a_src2, a_dst2, Ws1, Ws2, Wd, bd):
    B, N, F0 = h.shape
    c = jnp.float32(1.4426950408889634)
    as1 = a_src1[:, :, None] * c
    ad1 = a_dst1[:, None, :] * c
    as2 = a_src2[:, :, None] * c
    ad2 = a_dst2[:, None, :] * c
    WdR = Wd.reshape(_H, _D2)

    out = pl.pallas_call(
        _fused,
        grid=(B, 2, _H // 2),
        in_specs=[
            pl.BlockSpec((1, N, F0), lambda b, p, k: (b, 0, 0)),
            pl.BlockSpec((1, N, N), lambda b, p, k: (b, 0, 0)),
            pl.BlockSpec((_H, F0, _F1), lambda b, p, k: (0, 0, 0)),
            pl.BlockSpec((_H, _F1, 1), lambda b, p, k: (0, 0, 0)),
            pl.BlockSpec((_H, 1, _F1), lambda b, p, k: (0, 0, 0)),
            pl.BlockSpec((_H, _D1, _F2), lambda b, p, k: (0, 0, 0)),
            pl.BlockSpec((_H, _F2, 1), lambda b, p, k: (0, 0, 0)),
            pl.BlockSpec((_H, 1, _F2), lambda b, p, k: (0, 0, 0)),
            pl.BlockSpec((_D2, _D2), lambda b, p, k: (0, 0)),
            pl.BlockSpec((_H, _D2), lambda b, p, k: (0, 0)),
            pl.BlockSpec((_H, _D2), lambda b, p, k: (0, 0)),
        ],
        out_specs=pl.BlockSpec((1, 8, 128), lambda b, p, k: (b, 0, 0)),
        out_shape=jax.ShapeDtypeStruct((B, 8, 128), jnp.float32),
        compiler_params=pltpu.CompilerParams(
            dimension_semantics=("parallel", "arbitrary", "arbitrary")),
        scratch_shapes=[
            pltpu.VMEM((N, N), jnp.float32),
            pltpu.VMEM((_H, N, _F1), jnp.float32),
            pltpu.VMEM((_H, N, _F2), jnp.float32),
        ],
    )(h, adj, W1, as1, ad1, W2, as2, ad2, Ws1, Ws2, WdR)
    return out[:, 0, 0] + bd[0]
